# baseline (speedup 1.0000x reference)
import functools

import jax
import jax.numpy as jnp
from jax import lax
from jax.experimental import pallas as pl
from jax.experimental.pallas import tpu as pltpu
from jax.experimental.pallas import tpu_sc as plsc

N = 5000
NPAD = 5120
M = 50
MPAD = 64
B = 8
NC = 2
NS = 16
PER_TILE = NPAD // NS
VPT = PER_TILE // 16
B_PER_CORE = B // NC

_F32 = jnp.float32
_I32 = jnp.int32


def _softlog(p):
    bits = plsc.bitcast(p, _I32)
    e = (lax.shift_right_logical(bits, 23) & 255) - 127
    mbits = (bits & 0x007FFFFF) | 0x3F800000
    m = plsc.bitcast(mbits, _F32)
    big = m > jnp.float32(1.4142135)
    m = jnp.where(big, m * jnp.float32(0.5), m)
    ef = e.astype(_F32) + jnp.where(big, jnp.float32(1.0), jnp.float32(0.0))
    s = (m - jnp.float32(1.0)) / (m + jnp.float32(1.0))
    z = s * s
    poly = jnp.float32(2.0) + z * (
        jnp.float32(0.6666666) + z * (
            jnp.float32(0.4000001) + z * (
                jnp.float32(0.2857143) + z * jnp.float32(0.22222222))))
    return s * poly + ef * jnp.float32(0.69314718)


def _sc_body(a_h, c_h, g_h, n_h, t_h, out_h,
             anc, areaa, cls_cur, gxy, cnt, forced, colst,
             loc_col, accs, loc_acc, outv, sh_col, sh_acc):
    c = lax.axis_index("c")
    s = lax.axis_index("s")
    base = s * PER_TILE
    basef = base.astype(_F32)

    lane_i = jnp.arange(16, dtype=_I32)
    lane_f = lane_i.astype(_F32)
    lane0 = lane_i == 0
    ones16 = jnp.ones((16,), _F32)
    zeros16 = jnp.zeros((16,), _F32)

    for k in range(4):
        pltpu.sync_copy(a_h.at[pl.ds(k * NPAD + base, PER_TILE)],
                        anc.at[pl.ds(k * PER_TILE, PER_TILE)])
    for v in range(VPT):
        d = pl.ds(v * 16, 16)
        areaa[d] = ((anc[pl.ds(2 * PER_TILE + v * 16, 16)]
                     - anc[pl.ds(0 * PER_TILE + v * 16, 16)])
                    * (anc[pl.ds(3 * PER_TILE + v * 16, 16)]
                       - anc[pl.ds(1 * PER_TILE + v * 16, 16)]))

    pltpu.sync_copy(t_h, outv)
    thr_vec = outv[...]
    pltpu.sync_copy(n_h, accs)

    def batch_body(bi, acc):
        bb = c * B_PER_CORE + bi

        pltpu.sync_copy(g_h.at[pl.ds(bb * 4 * MPAD, 4 * MPAD)],
                        gxy.at[pl.ds(0, 4 * MPAD)])
        for k in range(2):
            pltpu.sync_copy(c_h.at[pl.ds((k * B + bb) * NPAD + base,
                                         PER_TILE)],
                            cls_cur.at[pl.ds(k * PER_TILE, PER_TILE)])

        for q in range(MPAD // 16):
            gx = gxy[pl.ds(0 * MPAD + q * 16, 16)]
            gy = gxy[pl.ds(1 * MPAD + q * 16, 16)]
            gw = gxy[pl.ds(2 * MPAD + q * 16, 16)] * jnp.float32(0.5)
            gh = gxy[pl.ds(3 * MPAD + q * 16, 16)] * jnp.float32(0.5)
            gxy[pl.ds(4 * MPAD + q * 16, 16)] = gx - gw
            gxy[pl.ds(5 * MPAD + q * 16, 16)] = gy - gh
            gxy[pl.ds(6 * MPAD + q * 16, 16)] = gx + gw
            gxy[pl.ds(7 * MPAD + q * 16, 16)] = gy + gh
            gxy[pl.ds(8 * MPAD + q * 16, 16)] = (gw + gw) * (gh + gh)

        for v in range(VPT):
            d = pl.ds(v * 16, 16)
            cnt[d] = zeros16
            forced[d] = zeros16

        bbv = jnp.full((16,), bb, _I32)
        nbv = plsc.load_gather(accs, [bbv])
        nbv = plsc.bitcast(nbv, _I32)

        def j_body(j, carry):
            jv = jnp.full((16,), j, _I32)
            validv = jv < nbv
            g0 = plsc.load_gather(gxy, [jv + jnp.int32(4 * MPAD)])
            g1 = plsc.load_gather(gxy, [jv + jnp.int32(5 * MPAD)])
            g2 = plsc.load_gather(gxy, [jv + jnp.int32(6 * MPAD)])
            g3 = plsc.load_gather(gxy, [jv + jnp.int32(7 * MPAD)])
            g4 = plsc.load_gather(gxy, [jv + jnp.int32(8 * MPAD)])
            vm = jnp.full((16,), jnp.float32(-1e30))
            vix = zeros16
            for v in range(VPT):
                d = pl.ds(v * 16, 16)
                ltx = jnp.maximum(anc[pl.ds(0 * PER_TILE + v * 16, 16)], g0)
                lty = jnp.maximum(anc[pl.ds(1 * PER_TILE + v * 16, 16)], g1)
                rbx = jnp.minimum(anc[pl.ds(2 * PER_TILE + v * 16, 16)], g2)
                rby = jnp.minimum(anc[pl.ds(3 * PER_TILE + v * 16, 16)], g3)
                iw = jnp.maximum(rbx - ltx, jnp.float32(0.0))
                ih = jnp.maximum(rby - lty, jnp.float32(0.0))
                inter = iw * ih
                union = areaa[d] + g4 - inter
                iou = inter / union
                hit = (iou > thr_vec) & validv
                plsc.addupdate(cnt.at[d],
                               jnp.where(hit, jnp.float32(1.0),
                                         jnp.float32(0.0)))
                bet = iou > vm
                vm = jnp.maximum(vm, iou)
                vix = jnp.where(bet, lane_f + jnp.float32(v * 16) + basef, vix)
            m = jnp.max(vm)
            cand = jnp.where(vm == m, vix, jnp.float32(4e9))
            mi = jnp.min(cand)
            plsc.store_scatter(colst, [jv], jnp.full((16,), m), mask=lane0)
            plsc.store_scatter(colst, [jv + jnp.int32(MPAD)],
                               jnp.full((16,), mi), mask=lane0)
            return carry

        lax.fori_loop(0, M, j_body, 0)

        pltpu.sync_copy(colst, sh_col.at[pl.ds((c * NS + s) * 2 * MPAD,
                                               2 * MPAD)])
        plsc.subcore_barrier()
        pltpu.sync_copy(sh_col.at[pl.ds(c * NS * 2 * MPAD, NS * 2 * MPAD)],
                        loc_col)
        plsc.subcore_barrier()

        for jc in range(MPAD // 16):
            gm = jnp.full((16,), jnp.float32(-1e30))
            gi = zeros16
            for r in range(NS):
                rm = loc_col[pl.ds(r * 2 * MPAD + jc * 16, 16)]
                bet = rm > gm
                gm = jnp.maximum(gm, rm)
                gi = jnp.where(bet,
                               loc_col[pl.ds(r * 2 * MPAD + MPAD + jc * 16,
                                             16)],
                               gi)
            jv16 = lane_i + jnp.int32(jc * 16)
            needfix = (jv16 < nbv) & (gm <= thr_vec)
            mine = needfix & (gi >= basef) & (gi < basef
                                              + jnp.float32(PER_TILE))
            li = (gi - basef).astype(_I32)
            plsc.store_scatter(forced, [li], ones16, mask=mine)
            c1v = plsc.load_gather(cls_cur, [li + jnp.int32(PER_TILE)],
                                   mask=mine)
            u1 = jnp.float32(1.0) - c1v
            f1 = -(u1 * u1) * _softlog(c1v)
            acc = acc + jnp.where(mine, jnp.float32(10.0) * f1, zeros16)

        for v in range(VPT):
            d = pl.ds(v * 16, 16)
            cv = cnt[d]
            pos = ((cv > jnp.float32(0.0))
                   | (forced[d] > jnp.float32(0.0)))
            p = jnp.where(pos, cls_cur[pl.ds(PER_TILE + v * 16, 16)],
                          cls_cur[pl.ds(v * 16, 16)])
            u = jnp.float32(1.0) - p
            f = -(u * u) * _softlog(p)
            acc = acc + f * (jnp.float32(1.0) + jnp.float32(10.0) * cv)
        return acc

    acc = lax.fori_loop(0, B_PER_CORE, batch_body, zeros16)

    accs[...] = acc
    pltpu.sync_copy(accs, sh_acc.at[pl.ds((c * NS + s) * 16, 16)])
    plsc.subcore_barrier()

    @pl.when(s == 0)
    def _():
        pltpu.sync_copy(sh_acc.at[pl.ds(c * NS * 16, NS * 16)], loc_acc)
        tot = jnp.zeros((16,), _F32)
        for r in range(NS):
            tot = tot + loc_acc[pl.ds(r * 16, 16)]
        ts = jnp.sum(tot) * jnp.float32(0.01 / B)
        outv[...] = jnp.full((16,), ts)
        pltpu.sync_copy(outv, out_h.at[pl.ds(c * 16, 16)])


@functools.partial(
    pl.kernel,
    out_type=jax.ShapeDtypeStruct((NC * 16,), _F32),
    mesh=plsc.VectorSubcoreMesh(core_axis_name="c", subcore_axis_name="s",
                                num_cores=NC, num_subcores=NS),
    compiler_params=pltpu.CompilerParams(needs_layout_passes=False),
    scratch_types=[
        pltpu.VMEM((4 * PER_TILE,), _F32),
        pltpu.VMEM((PER_TILE,), _F32),
        pltpu.VMEM((2 * PER_TILE,), _F32),
        pltpu.VMEM((9 * MPAD,), _F32),
        pltpu.VMEM((PER_TILE,), _F32),
        pltpu.VMEM((PER_TILE,), _F32),
        pltpu.VMEM((2 * MPAD,), _F32),
        pltpu.VMEM((NS * 2 * MPAD,), _F32),
        pltpu.VMEM((16,), _F32),
        pltpu.VMEM((NS * 16,), _F32),
        pltpu.VMEM((16,), _F32),
        pltpu.VMEM_SHARED((NC * NS * 2 * MPAD,), _F32),
        pltpu.VMEM_SHARED((NC * NS * 16,), _F32),
    ],
)
def _loss_sc(a_h, c_h, g_h, n_h, t_h, out_h, *scratch):
    _sc_body(a_h, c_h, g_h, n_h, t_h, out_h, *scratch)


def kernel(threshhold, batch_classes, anchors, batch_gt, batch_num_objects):
    a = jnp.zeros((4, NPAD), _F32).at[:, :N].set(anchors.T.astype(_F32))
    a = a.reshape(4 * NPAD)
    cme = jnp.transpose(batch_classes.astype(_F32), (2, 0, 1))
    cme = jnp.ones((2, B, NPAD), _F32).at[:, :, :N].set(cme)
    cme = cme.reshape(2 * B * NPAD)
    g = jnp.transpose(batch_gt.astype(_F32), (0, 2, 1))
    g = jnp.zeros((B, 4, MPAD), _F32).at[:, :, :M].set(g)
    g = g.reshape(B * 4 * MPAD)
    nf = jnp.zeros((16,), _I32).at[:B].set(batch_num_objects.astype(_I32))
    nf = lax.bitcast_convert_type(nf, _F32)
    tv = jnp.full((16,), threshhold, _F32)

    out = _loss_sc(a, cme, g, nf, tv)
    class_loss = out[:1] + out[16:17]
    coord_loss = jnp.zeros(1, _F32)
    total_loss = class_loss + coord_loss
    return (total_loss, class_loss, coord_loss)
